# baseline (speedup 1.0000x reference)
import jax
import jax.numpy as jnp
from jax.experimental import pallas as pl


_SEQ_BLOCK = 1024


def _body(pos_ref, out_ref):
    blk = pos_ref[...]
    for b in range(out_ref.shape[0]):
        out_ref[b] = blk


def kernel(x, pos_table):
    batch, seq_len = x.shape
    d_model = pos_table.shape[1]
    pos = pos_table[:seq_len]
    blk = _SEQ_BLOCK if seq_len % _SEQ_BLOCK == 0 else seq_len
    return pl.pallas_call(
        _body,
        grid=(seq_len // blk,),
        in_specs=[pl.BlockSpec((blk, d_model), lambda i: (i, 0))],
        out_specs=pl.BlockSpec((batch, blk, d_model), lambda i: (0, i, 0)),
        out_shape=jax.ShapeDtypeStruct((batch, seq_len, d_model), pos_table.dtype),
    )(pos)


# speedup vs baseline: 1.0509x; 1.0509x over previous
import jax
import jax.numpy as jnp
from jax.experimental import pallas as pl
from jax.experimental.pallas import tpu as pltpu


_N_CHUNKS = 8


def _copy_body(pos_hbm, out_hbm, buf, in_sems, out_sems):
    batch = out_hbm.shape[0]
    seq_len = pos_hbm.shape[0]
    chunk = seq_len // _N_CHUNKS

    def in_copy(c):
        rows = pl.ds(c * chunk, chunk)
        return pltpu.make_async_copy(pos_hbm.at[rows], buf.at[rows], in_sems.at[c])

    def out_copy(c, b):
        rows = pl.ds(c * chunk, chunk)
        return pltpu.make_async_copy(buf.at[rows], out_hbm.at[b, rows], out_sems.at[c, b])

    for c in range(_N_CHUNKS):
        in_copy(c).start()
    for c in range(_N_CHUNKS):
        in_copy(c).wait()
        for b in range(batch):
            out_copy(c, b).start()
    for c in range(_N_CHUNKS):
        for b in range(batch):
            out_copy(c, b).wait()


def kernel(x, pos_table):
    batch, seq_len = x.shape
    d_model = pos_table.shape[1]
    pos = pos_table[:seq_len]
    return pl.pallas_call(
        _copy_body,
        in_specs=[pl.BlockSpec(memory_space=pl.ANY)],
        out_specs=pl.BlockSpec(memory_space=pl.ANY),
        out_shape=jax.ShapeDtypeStruct((batch, seq_len, d_model), pos_table.dtype),
        scratch_shapes=[
            pltpu.VMEM((seq_len, d_model), pos_table.dtype),
            pltpu.SemaphoreType.DMA((_N_CHUNKS,)),
            pltpu.SemaphoreType.DMA((_N_CHUNKS, batch)),
        ],
    )(pos)
